# Initial kernel scaffold; baseline (speedup 1.0000x reference)
import jax
import jax.numpy as jnp
from jax.experimental import pallas as pl


def kernel(emb1, edge_index1, edge_attr1, emb2, edge_index2, edge_attr2, avg_v, g1_Wself, g1_Wmsg, g1_Wedge, g1_b, g1_Wcross, g2_Wself, g2_Wmsg, g2_Wedge, g2_b, g2_Wcross, map_M1, map_M2, cost_M1, cost_M2, att_W, ntn_W, ntn_V, ntn_b, reg_W1, reg_b1, reg_W2, reg_b2, reg_W3, reg_b3):
    raise NotImplementedError("write your pallas kernel here")


# speedup vs baseline: 3.3576x; 3.3576x over previous
import functools

import jax
import jax.numpy as jnp
from jax import lax
from jax.experimental import pallas as pl
from jax.experimental.pallas import tpu as pltpu
from jax.experimental.pallas import tpu_sc as plsc

F32 = jnp.float32


def _dot(a, b):
    return jnp.dot(a, b, preferred_element_type=F32)


def _dot_t(a, b):
    return lax.dot_general(a, b, (((1,), (1,)), ((), ())),
                           preferred_element_type=F32)


@functools.lru_cache(maxsize=None)
def _tc_pre(N, D, F, R):
    def body(e1, e2, wm, ws, xw1, xw2, sb1, sb2):
        xw1[...] = _dot(e1[...], wm[...])
        xw2[...] = _dot(e2[...], wm[...])
        sb1[...] = _dot(e1[...], ws[...])
        sb2[...] = _dot(e2[...], ws[...])

    out = tuple(jax.ShapeDtypeStruct((N, F), F32) for _ in range(4))
    row = lambda s: pl.BlockSpec((R, s), lambda i: (i, 0))
    full = lambda a, b: pl.BlockSpec((a, b), lambda i: (0, 0))
    return pl.pallas_call(
        body, grid=(N // R,),
        in_specs=[row(D), row(D), full(D, F), full(D, F)],
        out_specs=[row(F)] * 4, out_shape=out)


@functools.lru_cache(maxsize=None)
def _tc_edge(E, ED, F, R):
    def body(a1, a2, we, o1, o2):
        o1[...] = _dot(a1[...], we[...])
        o2[...] = _dot(a2[...], we[...])

    row = lambda s: pl.BlockSpec((R, s), lambda i: (i, 0))
    full = lambda a, b: pl.BlockSpec((a, b), lambda i: (0, 0))
    out = tuple(jax.ShapeDtypeStruct((E, F), F32) for _ in range(2))
    return pl.pallas_call(
        body, grid=(E // R,),
        in_specs=[row(ED), row(ED), full(ED, F)],
        out_specs=[row(F)] * 2, out_shape=out)


@functools.lru_cache(maxsize=None)
def _tc_relu_agg(N, F, R):
    def body(ax1, ax2, sb1, sb2, bb, h1r, h2r, cs1r, cs2r):
        h1 = jax.nn.relu((sb1[...] + ax1[...]) + bb[...])
        h2 = jax.nn.relu((sb2[...] + ax2[...]) + bb[...])
        h1r[...] = h1
        h2r[...] = h2

        @pl.when(pl.program_id(0) == 0)
        def _():
            cs1r[...] = jnp.zeros((1, F), F32)
            cs2r[...] = jnp.zeros((1, F), F32)

        cs1r[...] += jnp.sum(h1, axis=0, keepdims=True)
        cs2r[...] += jnp.sum(h2, axis=0, keepdims=True)

    row = lambda s: pl.BlockSpec((R, s), lambda i: (i, 0))
    full = lambda a, b: pl.BlockSpec((a, b), lambda i: (0, 0))
    out = (jax.ShapeDtypeStruct((N, F), F32),
           jax.ShapeDtypeStruct((N, F), F32),
           jax.ShapeDtypeStruct((1, F), F32),
           jax.ShapeDtypeStruct((1, F), F32))
    return pl.pallas_call(
        body, grid=(N // R,),
        in_specs=[row(F), row(F), row(F), row(F), full(1, F)],
        out_specs=[row(F), row(F), full(1, F), full(1, F)],
        out_shape=out)


@functools.lru_cache(maxsize=None)
def _tc_cross(N, F, FN, R, project):
    def body(*refs):
        if project:
            (h1, h2, cs1, cs2, wc, wm2, ws2,
             o1r, o2r, xw1r, xw2r, nb1r, nb2r) = refs
        else:
            (h1, h2, cs1, cs2, wc, o1r, o2r) = refs
        inv_n = F32(1.0 / N)
        t1 = jnp.tanh(_dot(cs2[...] * inv_n, wc[...]))
        t2 = jnp.tanh(_dot(cs1[...] * inv_n, wc[...]))
        o1 = h1[...] + t1
        o2 = h2[...] + t2
        o1r[...] = o1
        o2r[...] = o2
        if project:
            xw1r[...] = _dot(o1, wm2[...])
            xw2r[...] = _dot(o2, wm2[...])
            nb1r[...] = _dot(o1, ws2[...])
            nb2r[...] = _dot(o2, ws2[...])

    row = lambda s: pl.BlockSpec((R, s), lambda i: (i, 0))
    full = lambda a, b: pl.BlockSpec((a, b), lambda i: (0, 0))
    out = [jax.ShapeDtypeStruct((N, F), F32)] * 2
    in_specs = [row(F), row(F), full(1, F), full(1, F), full(F, F)]
    out_specs = [row(F), row(F)]
    if project:
        out += [jax.ShapeDtypeStruct((N, FN), F32)] * 4
        in_specs += [full(F, FN), full(F, FN)]
        out_specs += [row(FN)] * 4
    return pl.pallas_call(body, grid=(N // R,), in_specs=in_specs,
                          out_specs=out_specs, out_shape=tuple(out))


@functools.lru_cache(maxsize=None)
def _sc_conv(N, E, F):
    NS = 16
    EPT = E // NS
    CH = 128
    n_main = EPT // CH
    TAIL = EPT - n_main * CH
    RPT = (N // NS) // 8 * 8
    RTAIL = N - RPT * NS

    mesh = plsc.VectorSubcoreMesh(core_axis_name="c", subcore_axis_name="s",
                                  num_cores=2, num_subcores=16)

    outs = (jax.ShapeDtypeStruct((N, F), F32),
            jax.ShapeDtypeStruct((N, F), F32))

    scratch = [pltpu.VMEM_SHARED((N, F), F32),
               pltpu.VMEM((CH,), jnp.int32), pltpu.VMEM((CH,), jnp.int32),
               pltpu.VMEM((CH, F), F32), pltpu.VMEM((CH, F), F32)]
    if TAIL:
        scratch += [pltpu.VMEM((TAIL,), jnp.int32),
                    pltpu.VMEM((TAIL,), jnp.int32),
                    pltpu.VMEM((TAIL, F), F32), pltpu.VMEM((TAIL, F), F32)]
    scratch.append(pltpu.SemaphoreType.DMA)

    def body(*refs):
        it = iter(refs)
        xw1, xw2 = next(it), next(it)
        src1, dst1, ew1 = next(it), next(it), next(it)
        src2, dst2, ew2 = next(it), next(it), next(it)
        zx = next(it)
        aggx1, aggx2 = next(it), next(it)
        aggx_sp = next(it)
        idx_s, idx_d, rows, rows2 = next(it), next(it), next(it), next(it)
        if TAIL:
            idx_st, idx_dt, rows_t, rows2_t = (next(it), next(it), next(it),
                                               next(it))
        sem = next(it)

        c = lax.axis_index("c")
        s = lax.axis_index("s")
        r0 = s * RPT

        def sliced_copy(src_ref, dst_ref):
            pltpu.sync_copy(src_ref.at[pl.ds(r0, RPT)],
                            dst_ref.at[pl.ds(r0, RPT)])
            if RTAIL:
                @pl.when(s == 0)
                def _():
                    pltpu.sync_copy(src_ref.at[pl.ds(RPT * NS, RTAIL)],
                                    dst_ref.at[pl.ds(RPT * NS, RTAIL)])

        sliced_copy(zx, aggx_sp)
        plsc.subcore_barrier()

        def run(src, dst, ew, xw):
            base = s * EPT

            def chunk(off, isb, idb, rb, rb2, ch):
                pltpu.sync_copy(src.at[pl.ds(off, ch)], isb)
                pltpu.sync_copy(dst.at[pl.ds(off, ch)], idb)
                pltpu.async_copy(xw.at[isb], rb, sem).wait()
                pltpu.sync_copy(rb, aggx_sp.at[idb], add=True)
                pltpu.sync_copy(ew.at[pl.ds(off, ch)], rb2)
                pltpu.sync_copy(rb2, aggx_sp.at[idb], add=True)

            def bodyf(j, carry):
                chunk(base + j * CH, idx_s, idx_d, rows, rows2, CH)
                return carry

            lax.fori_loop(0, n_main, bodyf, 0)
            if TAIL:
                chunk(base + n_main * CH, idx_st, idx_dt, rows_t,
                      rows2_t, TAIL)

        @pl.when(c == 0)
        def _():
            run(src1, dst1, ew1, xw1)

        @pl.when(c == 1)
        def _():
            run(src2, dst2, ew2, xw2)

        plsc.subcore_barrier()

        @pl.when(c == 0)
        def _():
            sliced_copy(aggx_sp, aggx1)

        @pl.when(c == 1)
        def _():
            sliced_copy(aggx_sp, aggx2)

    return pl.kernel(body, out_type=outs, mesh=mesh,
                     scratch_types=tuple(scratch),
                     compiler_params=pltpu.CompilerParams(
                         use_tc_tiling_on_sc=False))


@functools.lru_cache(maxsize=None)
def _tc_matching(N, F1, F2, R):
    grid = (N // R,)

    def body(a1r, b1r, q1r, q2r, mm1, mm2, cm1, cm2, map_r, ssum_r):
        a1 = a1r[...]
        b1 = b1r[...]
        q1 = q1r[...]
        q2 = q2r[...]
        lm = _dot_t(_dot(a1, mm1[...]), q1) + _dot_t(_dot(b1, mm2[...]), q2)
        mx = jnp.max(lm, axis=1, keepdims=True)
        ex = jnp.exp(lm - mx)
        ssx = jnp.sum(ex, axis=1, keepdims=True)
        mapping = ex * (1.0 / ssx)
        map_r[...] = mapping
        lc = _dot_t(_dot(a1, cm1[...]), q1) + _dot_t(_dot(b1, cm2[...]), q2)
        partial = jnp.sum(mapping * lc)

        @pl.when(pl.program_id(0) == 0)
        def _():
            ssum_r[...] = jnp.zeros((1, 1), F32)

        ssum_r[...] += jnp.reshape(partial, (1, 1))

    return pl.pallas_call(
        body,
        grid=grid,
        in_specs=[
            pl.BlockSpec((R, F1), lambda i: (i, 0)),
            pl.BlockSpec((R, F2), lambda i: (i, 0)),
            pl.BlockSpec((N, F1), lambda i: (0, 0)),
            pl.BlockSpec((N, F2), lambda i: (0, 0)),
            pl.BlockSpec((F1, F1), lambda i: (0, 0)),
            pl.BlockSpec((F2, F2), lambda i: (0, 0)),
            pl.BlockSpec((F1, F1), lambda i: (0, 0)),
            pl.BlockSpec((F2, F2), lambda i: (0, 0)),
        ],
        out_specs=[
            pl.BlockSpec((R, N), lambda i: (i, 0)),
            pl.BlockSpec((1, 1), lambda i: (0, 0)),
        ],
        out_shape=[
            jax.ShapeDtypeStruct((N, N), F32),
            jax.ShapeDtypeStruct((1, 1), F32),
        ],
    )


@functools.lru_cache(maxsize=None)
def _tc_head(N, F2, T):
    def body(h1r, h2r, attw, ntnw, ntnv, ntnb, rw1, rb1, rw2, rb2, rw3, rb3,
             avgr, ssumr, score_r, pre_r):
        def attpool(h):
            m = jnp.mean(h, axis=0, keepdims=True)
            ctx = jnp.tanh(_dot(m, attw[...]))
            a = jax.nn.sigmoid(jnp.sum(h * ctx, axis=1, keepdims=True))
            return jnp.sum(h * a, axis=0, keepdims=True)

        ge1 = attpool(h1r[...])
        ge2 = attpool(h2r[...])
        kiota = lax.broadcasted_iota(jnp.int32, (1, T), 1)
        t1 = jnp.zeros((1, T), F32)
        for k in range(T):
            u = _dot(ge1, ntnw[k])
            sk = jnp.sum(u * ge2)
            t1 = t1 + jnp.where(kiota == k, sk, 0.0)
        cat = jnp.concatenate([ge1, ge2], axis=1)
        t2 = _dot(cat, ntnv[...]) + ntnb[...]
        sv = jax.nn.relu(t1 + t2)
        h = jax.nn.relu(_dot(sv, rw1[...]) + rb1[...])
        h = jax.nn.relu(_dot(h, rw2[...]) + rb2[...])
        bias = _dot(h, rw3[...]) + rb3[...]
        score = jax.nn.sigmoid(ssumr[...] + bias)
        score_r[...] = score
        pre_r[...] = -jnp.log(score) * avgr[...]

    out = (jax.ShapeDtypeStruct((1, 1), F32),
           jax.ShapeDtypeStruct((1, 1), F32))
    return pl.pallas_call(body, out_shape=out)


def kernel(emb1, edge_index1, edge_attr1, emb2, edge_index2, edge_attr2,
           avg_v, g1_Wself, g1_Wmsg, g1_Wedge, g1_b, g1_Wcross, g2_Wself,
           g2_Wmsg, g2_Wedge, g2_b, g2_Wcross, map_M1, map_M2, cost_M1,
           cost_M2, att_W, ntn_W, ntn_V, ntn_b, reg_W1, reg_b1, reg_W2,
           reg_b2, reg_W3, reg_b3):
    N, D = emb1.shape
    E = edge_index1.shape[1]
    ED = edge_attr1.shape[1]
    F1 = g1_Wself.shape[1]
    F2 = g2_Wself.shape[1]
    T = ntn_b.shape[0]

    src1 = edge_index1[0].astype(jnp.int32)
    dst1 = edge_index1[1].astype(jnp.int32)
    src2 = edge_index2[0].astype(jnp.int32)
    dst2 = edge_index2[1].astype(jnp.int32)
    ea1 = edge_attr1.astype(F32)
    ea2 = edge_attr2.astype(F32)

    zx1 = jnp.zeros((N, F1), F32)
    zx2 = jnp.zeros((N, F2), F32)

    RB = 1000 if N % 1000 == 0 else N
    RE = 8000 if E % 8000 == 0 else E

    xw1a, xw1b, sb1a, sb1b = _tc_pre(N, D, F1, RB)(
        emb1, emb2, g1_Wmsg, g1_Wself)
    ew1a, ew1b = _tc_edge(E, ED, F1, RE)(ea1, ea2, g1_Wedge)
    ax1a, ax1b = _sc_conv(N, E, F1)(
        xw1a, xw1b, src1, dst1, ew1a, src2, dst2, ew1b, zx1)
    h1a, h1b, cs1a, cs1b = _tc_relu_agg(N, F1, RB)(
        ax1a, ax1b, sb1a, sb1b, g1_b.reshape(1, F1))
    e1a, e2a, xw2a, xw2b, sb2a, sb2b = _tc_cross(N, F1, F2, RB, True)(
        h1a, h1b, cs1a, cs1b, g1_Wcross, g2_Wmsg, g2_Wself)

    ew2a, ew2b = _tc_edge(E, ED, F2, RE)(ea1, ea2, g2_Wedge)
    ax2a, ax2b = _sc_conv(N, E, F2)(
        xw2a, xw2b, src1, dst1, ew2a, src2, dst2, ew2b, zx2)
    h2a, h2b, cs2a, cs2b = _tc_relu_agg(N, F2, RB)(
        ax2a, ax2b, sb2a, sb2b, g2_b.reshape(1, F2))
    e1b, e2b = _tc_cross(N, F2, 0, RB, False)(
        h2a, h2b, cs2a, cs2b, g2_Wcross)

    mapping, ssum = _tc_matching(N, F1, F2, 80)(
        e1a, e1b, e2a, e2b, map_M1, map_M2, cost_M1, cost_M2)

    score, pre = _tc_head(N, F2, T)(
        e1b, e2b, att_W, jnp.transpose(ntn_W, (2, 0, 1)),
        jnp.transpose(ntn_V), ntn_b.reshape(1, T), reg_W1,
        reg_b1.reshape(1, -1), reg_W2, reg_b2.reshape(1, -1), reg_W3,
        reg_b3.reshape(1, -1), avg_v.reshape(1, 1).astype(F32), ssum)

    return score.reshape(()), pre.reshape((1,)), mapping


# speedup vs baseline: 4.0865x; 1.2171x over previous
import functools

import jax
import jax.numpy as jnp
from jax import lax
from jax.experimental import pallas as pl
from jax.experimental.pallas import tpu as pltpu
from jax.experimental.pallas import tpu_sc as plsc

F32 = jnp.float32


def _dot(a, b):
    return jnp.dot(a, b, preferred_element_type=F32)


def _dot_t(a, b):
    return lax.dot_general(a, b, (((1,), (1,)), ((), ())),
                           preferred_element_type=F32)


@functools.lru_cache(maxsize=None)
def _tc_pre(N, D, F, R):
    def body(e1, e2, wm, ws, xw1, xw2, sb1, sb2):
        xw1[...] = _dot(e1[...], wm[...])
        xw2[...] = _dot(e2[...], wm[...])
        sb1[...] = _dot(e1[...], ws[...])
        sb2[...] = _dot(e2[...], ws[...])

    out = tuple(jax.ShapeDtypeStruct((N, F), F32) for _ in range(4))
    row = lambda s: pl.BlockSpec((R, s), lambda i: (i, 0))
    full = lambda a, b: pl.BlockSpec((a, b), lambda i: (0, 0))
    return pl.pallas_call(
        body, grid=(N // R,),
        in_specs=[row(D), row(D), full(D, F), full(D, F)],
        out_specs=[row(F)] * 4, out_shape=out)


@functools.lru_cache(maxsize=None)
def _tc_edge(E, ED, F, R):
    def body(a1, a2, we, o1, o2):
        o1[...] = _dot(a1[...], we[...])
        o2[...] = _dot(a2[...], we[...])

    row = lambda s: pl.BlockSpec((R, s), lambda i: (i, 0))
    full = lambda a, b: pl.BlockSpec((a, b), lambda i: (0, 0))
    out = tuple(jax.ShapeDtypeStruct((E, F), F32) for _ in range(2))
    return pl.pallas_call(
        body, grid=(E // R,),
        in_specs=[row(ED), row(ED), full(ED, F)],
        out_specs=[row(F)] * 2, out_shape=out)


@functools.lru_cache(maxsize=None)
def _tc_relu_agg(N, F, R):
    def body(ax1, ax2, sb1, sb2, bb, h1r, h2r, cs1r, cs2r):
        h1 = jax.nn.relu((sb1[...] + ax1[...]) + bb[...])
        h2 = jax.nn.relu((sb2[...] + ax2[...]) + bb[...])
        h1r[...] = h1
        h2r[...] = h2

        @pl.when(pl.program_id(0) == 0)
        def _():
            cs1r[...] = jnp.zeros((1, F), F32)
            cs2r[...] = jnp.zeros((1, F), F32)

        cs1r[...] += jnp.sum(h1, axis=0, keepdims=True)
        cs2r[...] += jnp.sum(h2, axis=0, keepdims=True)

    row = lambda s: pl.BlockSpec((R, s), lambda i: (i, 0))
    full = lambda a, b: pl.BlockSpec((a, b), lambda i: (0, 0))
    out = (jax.ShapeDtypeStruct((N, F), F32),
           jax.ShapeDtypeStruct((N, F), F32),
           jax.ShapeDtypeStruct((1, F), F32),
           jax.ShapeDtypeStruct((1, F), F32))
    return pl.pallas_call(
        body, grid=(N // R,),
        in_specs=[row(F), row(F), row(F), row(F), full(1, F)],
        out_specs=[row(F), row(F), full(1, F), full(1, F)],
        out_shape=out)


@functools.lru_cache(maxsize=None)
def _tc_cross(N, F, FN, R, project):
    def body(*refs):
        if project:
            (h1, h2, cs1, cs2, wc, wm2, ws2,
             o1r, o2r, xw1r, xw2r, nb1r, nb2r) = refs
        else:
            (h1, h2, cs1, cs2, wc, o1r, o2r) = refs
        inv_n = F32(1.0 / N)
        t1 = jnp.tanh(_dot(cs2[...] * inv_n, wc[...]))
        t2 = jnp.tanh(_dot(cs1[...] * inv_n, wc[...]))
        o1 = h1[...] + t1
        o2 = h2[...] + t2
        o1r[...] = o1
        o2r[...] = o2
        if project:
            xw1r[...] = _dot(o1, wm2[...])
            xw2r[...] = _dot(o2, wm2[...])
            nb1r[...] = _dot(o1, ws2[...])
            nb2r[...] = _dot(o2, ws2[...])

    row = lambda s: pl.BlockSpec((R, s), lambda i: (i, 0))
    full = lambda a, b: pl.BlockSpec((a, b), lambda i: (0, 0))
    out = [jax.ShapeDtypeStruct((N, F), F32)] * 2
    in_specs = [row(F), row(F), full(1, F), full(1, F), full(F, F)]
    out_specs = [row(F), row(F)]
    if project:
        out += [jax.ShapeDtypeStruct((N, FN), F32)] * 4
        in_specs += [full(F, FN), full(F, FN)]
        out_specs += [row(FN)] * 4
    return pl.pallas_call(body, grid=(N // R,), in_specs=in_specs,
                          out_specs=out_specs, out_shape=tuple(out))


@functools.lru_cache(maxsize=None)
def _sc_conv(N, E, F):
    NS = 16
    EPT = E // NS
    CH = 128
    n_main = EPT // CH
    TAIL = EPT - n_main * CH
    RPT = (N // NS) // 8 * 8
    RTAIL = N - RPT * NS

    NBUF = 4
    n_grp = n_main // NBUF
    n_rem = n_main - n_grp * NBUF

    mesh = plsc.VectorSubcoreMesh(core_axis_name="c", subcore_axis_name="s",
                                  num_cores=2, num_subcores=16)

    outs = (jax.ShapeDtypeStruct((N, F), F32),
            jax.ShapeDtypeStruct((N, F), F32))

    scratch = [pltpu.VMEM_SHARED((N, F), F32)]
    for _ in range(NBUF):
        scratch += [pltpu.VMEM((CH,), jnp.int32), pltpu.VMEM((CH,), jnp.int32),
                    pltpu.VMEM((CH, F), F32), pltpu.VMEM((CH, F), F32)]
    if TAIL:
        scratch += [pltpu.VMEM((TAIL,), jnp.int32),
                    pltpu.VMEM((TAIL,), jnp.int32),
                    pltpu.VMEM((TAIL, F), F32), pltpu.VMEM((TAIL, F), F32)]
    scratch += [pltpu.SemaphoreType.DMA] * (2 * NBUF)

    def body(*refs):
        it = iter(refs)
        xw1, xw2 = next(it), next(it)
        src1, dst1, ew1 = next(it), next(it), next(it)
        src2, dst2, ew2 = next(it), next(it), next(it)
        zx = next(it)
        aggx1, aggx2 = next(it), next(it)
        aggx_sp = next(it)
        bufs = [(next(it), next(it), next(it), next(it))
                for _ in range(NBUF)]
        if TAIL:
            tailb = (next(it), next(it), next(it), next(it))
        sems = [(next(it), next(it)) for _ in range(NBUF)]

        c = lax.axis_index("c")
        s = lax.axis_index("s")
        r0 = s * RPT

        def sliced_copy(src_ref, dst_ref):
            pltpu.sync_copy(src_ref.at[pl.ds(r0, RPT)],
                            dst_ref.at[pl.ds(r0, RPT)])
            if RTAIL:
                @pl.when(s == 0)
                def _():
                    pltpu.sync_copy(src_ref.at[pl.ds(RPT * NS, RTAIL)],
                                    dst_ref.at[pl.ds(RPT * NS, RTAIL)])

        sliced_copy(zx, aggx_sp)
        plsc.subcore_barrier()

        def run(src, dst, ew, xw):
            base = s * EPT

            def start(off, buf, sem2, ch):
                isb, idb, rb, rb2 = buf
                sg, se = sem2
                pltpu.sync_copy(src.at[pl.ds(off, ch)], isb)
                pltpu.sync_copy(dst.at[pl.ds(off, ch)], idb)
                hg = pltpu.async_copy(xw.at[isb], rb, sg)
                he = pltpu.async_copy(ew.at[pl.ds(off, ch)], rb2, se)
                return hg, he

            def finish(hs, buf):
                hg, he = hs
                isb, idb, rb, rb2 = buf
                hg.wait()
                pltpu.sync_copy(rb, aggx_sp.at[idb], add=True)
                he.wait()
                pltpu.sync_copy(rb2, aggx_sp.at[idb], add=True)

            def bodyf(g, carry):
                goff = base + g * (NBUF * CH)
                hs = [start(goff + b * CH, bufs[b], sems[b], CH)
                      for b in range(NBUF)]
                for b in range(NBUF):
                    finish(hs[b], bufs[b])
                return carry

            lax.fori_loop(0, n_grp, bodyf, 0)
            for j in range(n_rem):
                off = base + (n_grp * NBUF + j) * CH
                finish(start(off, bufs[0], sems[0], CH), bufs[0])
            if TAIL:
                finish(start(base + n_main * CH, tailb, sems[0], TAIL),
                       tailb)

        @pl.when(c == 0)
        def _():
            run(src1, dst1, ew1, xw1)

        @pl.when(c == 1)
        def _():
            run(src2, dst2, ew2, xw2)

        plsc.subcore_barrier()

        @pl.when(c == 0)
        def _():
            sliced_copy(aggx_sp, aggx1)

        @pl.when(c == 1)
        def _():
            sliced_copy(aggx_sp, aggx2)

    return pl.kernel(body, out_type=outs, mesh=mesh,
                     scratch_types=tuple(scratch),
                     compiler_params=pltpu.CompilerParams(
                         use_tc_tiling_on_sc=False))


@functools.lru_cache(maxsize=None)
def _tc_matching(N, F1, F2, R):
    grid = (N // R,)

    def body(a1r, b1r, q1r, q2r, mm1, mm2, cm1, cm2, map_r, ssum_r):
        a1 = a1r[...]
        b1 = b1r[...]
        q1 = q1r[...]
        q2 = q2r[...]
        lm = _dot_t(_dot(a1, mm1[...]), q1) + _dot_t(_dot(b1, mm2[...]), q2)
        mx = jnp.max(lm, axis=1, keepdims=True)
        ex = jnp.exp(lm - mx)
        ssx = jnp.sum(ex, axis=1, keepdims=True)
        mapping = ex * (1.0 / ssx)
        map_r[...] = mapping
        lc = _dot_t(_dot(a1, cm1[...]), q1) + _dot_t(_dot(b1, cm2[...]), q2)
        partial = jnp.sum(mapping * lc)

        @pl.when(pl.program_id(0) == 0)
        def _():
            ssum_r[...] = jnp.zeros((1, 1), F32)

        ssum_r[...] += jnp.reshape(partial, (1, 1))

    return pl.pallas_call(
        body,
        grid=grid,
        in_specs=[
            pl.BlockSpec((R, F1), lambda i: (i, 0)),
            pl.BlockSpec((R, F2), lambda i: (i, 0)),
            pl.BlockSpec((N, F1), lambda i: (0, 0)),
            pl.BlockSpec((N, F2), lambda i: (0, 0)),
            pl.BlockSpec((F1, F1), lambda i: (0, 0)),
            pl.BlockSpec((F2, F2), lambda i: (0, 0)),
            pl.BlockSpec((F1, F1), lambda i: (0, 0)),
            pl.BlockSpec((F2, F2), lambda i: (0, 0)),
        ],
        out_specs=[
            pl.BlockSpec((R, N), lambda i: (i, 0)),
            pl.BlockSpec((1, 1), lambda i: (0, 0)),
        ],
        out_shape=[
            jax.ShapeDtypeStruct((N, N), F32),
            jax.ShapeDtypeStruct((1, 1), F32),
        ],
    )


@functools.lru_cache(maxsize=None)
def _tc_head(N, F2, T):
    def body(h1r, h2r, attw, ntnw, ntnv, ntnb, rw1, rb1, rw2, rb2, rw3, rb3,
             avgr, ssumr, score_r, pre_r):
        def attpool(h):
            m = jnp.mean(h, axis=0, keepdims=True)
            ctx = jnp.tanh(_dot(m, attw[...]))
            a = jax.nn.sigmoid(jnp.sum(h * ctx, axis=1, keepdims=True))
            return jnp.sum(h * a, axis=0, keepdims=True)

        ge1 = attpool(h1r[...])
        ge2 = attpool(h2r[...])
        kiota = lax.broadcasted_iota(jnp.int32, (1, T), 1)
        t1 = jnp.zeros((1, T), F32)
        for k in range(T):
            u = _dot(ge1, ntnw[k])
            sk = jnp.sum(u * ge2)
            t1 = t1 + jnp.where(kiota == k, sk, 0.0)
        cat = jnp.concatenate([ge1, ge2], axis=1)
        t2 = _dot(cat, ntnv[...]) + ntnb[...]
        sv = jax.nn.relu(t1 + t2)
        h = jax.nn.relu(_dot(sv, rw1[...]) + rb1[...])
        h = jax.nn.relu(_dot(h, rw2[...]) + rb2[...])
        bias = _dot(h, rw3[...]) + rb3[...]
        score = jax.nn.sigmoid(ssumr[...] + bias)
        score_r[...] = score
        pre_r[...] = -jnp.log(score) * avgr[...]

    out = (jax.ShapeDtypeStruct((1, 1), F32),
           jax.ShapeDtypeStruct((1, 1), F32))
    return pl.pallas_call(body, out_shape=out)


def kernel(emb1, edge_index1, edge_attr1, emb2, edge_index2, edge_attr2,
           avg_v, g1_Wself, g1_Wmsg, g1_Wedge, g1_b, g1_Wcross, g2_Wself,
           g2_Wmsg, g2_Wedge, g2_b, g2_Wcross, map_M1, map_M2, cost_M1,
           cost_M2, att_W, ntn_W, ntn_V, ntn_b, reg_W1, reg_b1, reg_W2,
           reg_b2, reg_W3, reg_b3):
    N, D = emb1.shape
    E = edge_index1.shape[1]
    ED = edge_attr1.shape[1]
    F1 = g1_Wself.shape[1]
    F2 = g2_Wself.shape[1]
    T = ntn_b.shape[0]

    src1 = edge_index1[0].astype(jnp.int32)
    dst1 = edge_index1[1].astype(jnp.int32)
    src2 = edge_index2[0].astype(jnp.int32)
    dst2 = edge_index2[1].astype(jnp.int32)
    ea1 = edge_attr1.astype(F32)
    ea2 = edge_attr2.astype(F32)

    zx1 = jnp.zeros((N, F1), F32)
    zx2 = jnp.zeros((N, F2), F32)

    RB = 1000 if N % 1000 == 0 else N
    RE = 8000 if E % 8000 == 0 else E

    xw1a, xw1b, sb1a, sb1b = _tc_pre(N, D, F1, RB)(
        emb1, emb2, g1_Wmsg, g1_Wself)
    ew1a, ew1b = _tc_edge(E, ED, F1, RE)(ea1, ea2, g1_Wedge)
    ax1a, ax1b = _sc_conv(N, E, F1)(
        xw1a, xw1b, src1, dst1, ew1a, src2, dst2, ew1b, zx1)
    h1a, h1b, cs1a, cs1b = _tc_relu_agg(N, F1, RB)(
        ax1a, ax1b, sb1a, sb1b, g1_b.reshape(1, F1))
    e1a, e2a, xw2a, xw2b, sb2a, sb2b = _tc_cross(N, F1, F2, RB, True)(
        h1a, h1b, cs1a, cs1b, g1_Wcross, g2_Wmsg, g2_Wself)

    ew2a, ew2b = _tc_edge(E, ED, F2, RE)(ea1, ea2, g2_Wedge)
    ax2a, ax2b = _sc_conv(N, E, F2)(
        xw2a, xw2b, src1, dst1, ew2a, src2, dst2, ew2b, zx2)
    h2a, h2b, cs2a, cs2b = _tc_relu_agg(N, F2, RB)(
        ax2a, ax2b, sb2a, sb2b, g2_b.reshape(1, F2))
    e1b, e2b = _tc_cross(N, F2, 0, RB, False)(
        h2a, h2b, cs2a, cs2b, g2_Wcross)

    mapping, ssum = _tc_matching(N, F1, F2, 80)(
        e1a, e1b, e2a, e2b, map_M1, map_M2, cost_M1, cost_M2)

    score, pre = _tc_head(N, F2, T)(
        e1b, e2b, att_W, jnp.transpose(ntn_W, (2, 0, 1)),
        jnp.transpose(ntn_V), ntn_b.reshape(1, T), reg_W1,
        reg_b1.reshape(1, -1), reg_W2, reg_b2.reshape(1, -1), reg_W3,
        reg_b3.reshape(1, -1), avg_v.reshape(1, 1).astype(F32), ssum)

    return score.reshape(()), pre.reshape((1,)), mapping


# speedup vs baseline: 4.4423x; 1.0871x over previous
import functools

import jax
import jax.numpy as jnp
from jax import lax
from jax.experimental import pallas as pl
from jax.experimental.pallas import tpu as pltpu
from jax.experimental.pallas import tpu_sc as plsc

F32 = jnp.float32


def _dot(a, b):
    return jnp.dot(a, b, preferred_element_type=F32)


def _dot_t(a, b):
    return lax.dot_general(a, b, (((1,), (1,)), ((), ())),
                           preferred_element_type=F32)


@functools.lru_cache(maxsize=None)
def _tc_pre(N, D, F, R):
    def body(e1, e2, wm, ws, xw1, xw2, sb1, sb2):
        xw1[...] = _dot(e1[...], wm[...])
        xw2[...] = _dot(e2[...], wm[...])
        sb1[...] = _dot(e1[...], ws[...])
        sb2[...] = _dot(e2[...], ws[...])

    out = tuple(jax.ShapeDtypeStruct((N, F), F32) for _ in range(4))
    row = lambda s: pl.BlockSpec((R, s), lambda i: (i, 0))
    full = lambda a, b: pl.BlockSpec((a, b), lambda i: (0, 0))
    return pl.pallas_call(
        body, grid=(N // R,),
        in_specs=[row(D), row(D), full(D, F), full(D, F)],
        out_specs=[row(F)] * 4, out_shape=out)


@functools.lru_cache(maxsize=None)
def _tc_edge(E, ED, F, R):
    def body(a1, a2, we, o1, o2):
        o1[...] = _dot(a1[...], we[...])
        o2[...] = _dot(a2[...], we[...])

    row = lambda s: pl.BlockSpec((R, s), lambda i: (i, 0))
    full = lambda a, b: pl.BlockSpec((a, b), lambda i: (0, 0))
    out = tuple(jax.ShapeDtypeStruct((E, F), F32) for _ in range(2))
    return pl.pallas_call(
        body, grid=(E // R,),
        in_specs=[row(ED), row(ED), full(ED, F)],
        out_specs=[row(F)] * 2, out_shape=out)


@functools.lru_cache(maxsize=None)
def _tc_relu_agg(N, F, R):
    def body(ax1, ax2, sb1, sb2, bb, h1r, h2r, cs1r, cs2r):
        h1 = jax.nn.relu((sb1[...] + ax1[...]) + bb[...])
        h2 = jax.nn.relu((sb2[...] + ax2[...]) + bb[...])
        h1r[...] = h1
        h2r[...] = h2

        @pl.when(pl.program_id(0) == 0)
        def _():
            cs1r[...] = jnp.zeros((1, F), F32)
            cs2r[...] = jnp.zeros((1, F), F32)

        cs1r[...] += jnp.sum(h1, axis=0, keepdims=True)
        cs2r[...] += jnp.sum(h2, axis=0, keepdims=True)

    row = lambda s: pl.BlockSpec((R, s), lambda i: (i, 0))
    full = lambda a, b: pl.BlockSpec((a, b), lambda i: (0, 0))
    out = (jax.ShapeDtypeStruct((N, F), F32),
           jax.ShapeDtypeStruct((N, F), F32),
           jax.ShapeDtypeStruct((1, F), F32),
           jax.ShapeDtypeStruct((1, F), F32))
    return pl.pallas_call(
        body, grid=(N // R,),
        in_specs=[row(F), row(F), row(F), row(F), full(1, F)],
        out_specs=[row(F), row(F), full(1, F), full(1, F)],
        out_shape=out)


@functools.lru_cache(maxsize=None)
def _tc_cross(N, F, FN, R, project):
    def body(*refs):
        if project:
            (h1, h2, cs1, cs2, wc, wm2, ws2,
             o1r, o2r, xw1r, xw2r, nb1r, nb2r) = refs
        else:
            (h1, h2, cs1, cs2, wc, o1r, o2r) = refs
        inv_n = F32(1.0 / N)
        t1 = jnp.tanh(_dot(cs2[...] * inv_n, wc[...]))
        t2 = jnp.tanh(_dot(cs1[...] * inv_n, wc[...]))
        o1 = h1[...] + t1
        o2 = h2[...] + t2
        o1r[...] = o1
        o2r[...] = o2
        if project:
            xw1r[...] = _dot(o1, wm2[...])
            xw2r[...] = _dot(o2, wm2[...])
            nb1r[...] = _dot(o1, ws2[...])
            nb2r[...] = _dot(o2, ws2[...])

    row = lambda s: pl.BlockSpec((R, s), lambda i: (i, 0))
    full = lambda a, b: pl.BlockSpec((a, b), lambda i: (0, 0))
    out = [jax.ShapeDtypeStruct((N, F), F32)] * 2
    in_specs = [row(F), row(F), full(1, F), full(1, F), full(F, F)]
    out_specs = [row(F), row(F)]
    if project:
        out += [jax.ShapeDtypeStruct((N, FN), F32)] * 4
        in_specs += [full(F, FN), full(F, FN)]
        out_specs += [row(FN)] * 4
    return pl.pallas_call(body, grid=(N // R,), in_specs=in_specs,
                          out_specs=out_specs, out_shape=tuple(out))


@functools.lru_cache(maxsize=None)
def _sc_conv(N, E, F):
    NS = 16
    EPT = E // NS
    CH = 128
    n_main = EPT // CH
    TAIL = EPT - n_main * CH
    RPT = (N // NS) // 8 * 8
    RTAIL = N - RPT * NS

    NBUF = 5
    n_grp = n_main // NBUF
    n_rem = n_main - n_grp * NBUF

    mesh = plsc.VectorSubcoreMesh(core_axis_name="c", subcore_axis_name="s",
                                  num_cores=2, num_subcores=16)

    outs = (jax.ShapeDtypeStruct((N, F), F32),
            jax.ShapeDtypeStruct((N, F), F32))

    scratch = [pltpu.VMEM_SHARED((N, F), F32)]
    for _ in range(NBUF):
        scratch += [pltpu.VMEM((CH,), jnp.int32), pltpu.VMEM((CH,), jnp.int32),
                    pltpu.VMEM((CH, F), F32), pltpu.VMEM((CH, F), F32)]
    if TAIL:
        scratch += [pltpu.VMEM((TAIL,), jnp.int32),
                    pltpu.VMEM((TAIL,), jnp.int32),
                    pltpu.VMEM((TAIL, F), F32), pltpu.VMEM((TAIL, F), F32)]
    scratch += [pltpu.SemaphoreType.DMA] * (2 * NBUF)

    def body(*refs):
        it = iter(refs)
        xw1, xw2 = next(it), next(it)
        src1, dst1, ew1 = next(it), next(it), next(it)
        src2, dst2, ew2 = next(it), next(it), next(it)
        zx = next(it)
        aggx1, aggx2 = next(it), next(it)
        aggx_sp = next(it)
        bufs = [(next(it), next(it), next(it), next(it))
                for _ in range(NBUF)]
        if TAIL:
            tailb = (next(it), next(it), next(it), next(it))
        sems = [(next(it), next(it)) for _ in range(NBUF)]

        c = lax.axis_index("c")
        s = lax.axis_index("s")
        r0 = s * RPT

        def sliced_copy(src_ref, dst_ref):
            pltpu.sync_copy(src_ref.at[pl.ds(r0, RPT)],
                            dst_ref.at[pl.ds(r0, RPT)])
            if RTAIL:
                @pl.when(s == 0)
                def _():
                    pltpu.sync_copy(src_ref.at[pl.ds(RPT * NS, RTAIL)],
                                    dst_ref.at[pl.ds(RPT * NS, RTAIL)])

        sliced_copy(zx, aggx_sp)
        plsc.subcore_barrier()

        def run(src, dst, ew, xw):
            base = s * EPT

            def start(off, buf, sem2, ch):
                isb, idb, rb, rb2 = buf
                sg, se = sem2
                pltpu.sync_copy(src.at[pl.ds(off, ch)], isb)
                pltpu.sync_copy(dst.at[pl.ds(off, ch)], idb)
                hg = pltpu.async_copy(xw.at[isb], rb, sg)
                he = pltpu.async_copy(ew.at[pl.ds(off, ch)], rb2, se)
                return hg, he

            def finish(hs, buf):
                hg, he = hs
                isb, idb, rb, rb2 = buf
                hg.wait()
                pltpu.sync_copy(rb, aggx_sp.at[idb], add=True)
                he.wait()
                pltpu.sync_copy(rb2, aggx_sp.at[idb], add=True)

            def bodyf(g, carry):
                goff = base + g * (NBUF * CH)
                hs = [start(goff + b * CH, bufs[b], sems[b], CH)
                      for b in range(NBUF)]
                for b in range(NBUF):
                    finish(hs[b], bufs[b])
                return carry

            lax.fori_loop(0, n_grp, bodyf, 0)
            for j in range(n_rem):
                off = base + (n_grp * NBUF + j) * CH
                finish(start(off, bufs[0], sems[0], CH), bufs[0])
            if TAIL:
                finish(start(base + n_main * CH, tailb, sems[0], TAIL),
                       tailb)

        @pl.when(c == 0)
        def _():
            run(src1, dst1, ew1, xw1)

        @pl.when(c == 1)
        def _():
            run(src2, dst2, ew2, xw2)

        plsc.subcore_barrier()

        @pl.when(c == 0)
        def _():
            sliced_copy(aggx_sp, aggx1)

        @pl.when(c == 1)
        def _():
            sliced_copy(aggx_sp, aggx2)

    return pl.kernel(body, out_type=outs, mesh=mesh,
                     scratch_types=tuple(scratch),
                     compiler_params=pltpu.CompilerParams(
                         use_tc_tiling_on_sc=False))


@functools.lru_cache(maxsize=None)
def _tc_matching(N, F1, F2, R):
    grid = (N // R,)

    def body(a1r, b1r, q1r, q2r, mm1, mm2, cm1, cm2, map_r, ssum_r):
        a1 = a1r[...]
        b1 = b1r[...]
        q1 = q1r[...]
        q2 = q2r[...]
        lm = _dot_t(_dot(a1, mm1[...]), q1) + _dot_t(_dot(b1, mm2[...]), q2)
        mx = jnp.max(lm, axis=1, keepdims=True)
        ex = jnp.exp(lm - mx)
        ssx = jnp.sum(ex, axis=1, keepdims=True)
        mapping = ex * (1.0 / ssx)
        map_r[...] = mapping
        lc = _dot_t(_dot(a1, cm1[...]), q1) + _dot_t(_dot(b1, cm2[...]), q2)
        partial = jnp.sum(mapping * lc)

        @pl.when(pl.program_id(0) == 0)
        def _():
            ssum_r[...] = jnp.zeros((1, 1), F32)

        ssum_r[...] += jnp.reshape(partial, (1, 1))

    return pl.pallas_call(
        body,
        grid=grid,
        in_specs=[
            pl.BlockSpec((R, F1), lambda i: (i, 0)),
            pl.BlockSpec((R, F2), lambda i: (i, 0)),
            pl.BlockSpec((N, F1), lambda i: (0, 0)),
            pl.BlockSpec((N, F2), lambda i: (0, 0)),
            pl.BlockSpec((F1, F1), lambda i: (0, 0)),
            pl.BlockSpec((F2, F2), lambda i: (0, 0)),
            pl.BlockSpec((F1, F1), lambda i: (0, 0)),
            pl.BlockSpec((F2, F2), lambda i: (0, 0)),
        ],
        out_specs=[
            pl.BlockSpec((R, N), lambda i: (i, 0)),
            pl.BlockSpec((1, 1), lambda i: (0, 0)),
        ],
        out_shape=[
            jax.ShapeDtypeStruct((N, N), F32),
            jax.ShapeDtypeStruct((1, 1), F32),
        ],
    )


@functools.lru_cache(maxsize=None)
def _tc_head(N, F2, T):
    def body(h1r, h2r, attw, ntnw, ntnv, ntnb, rw1, rb1, rw2, rb2, rw3, rb3,
             avgr, ssumr, score_r, pre_r):
        def attpool(h):
            m = jnp.mean(h, axis=0, keepdims=True)
            ctx = jnp.tanh(_dot(m, attw[...]))
            a = jax.nn.sigmoid(jnp.sum(h * ctx, axis=1, keepdims=True))
            return jnp.sum(h * a, axis=0, keepdims=True)

        ge1 = attpool(h1r[...])
        ge2 = attpool(h2r[...])
        kiota = lax.broadcasted_iota(jnp.int32, (1, T), 1)
        t1 = jnp.zeros((1, T), F32)
        for k in range(T):
            u = _dot(ge1, ntnw[k])
            sk = jnp.sum(u * ge2)
            t1 = t1 + jnp.where(kiota == k, sk, 0.0)
        cat = jnp.concatenate([ge1, ge2], axis=1)
        t2 = _dot(cat, ntnv[...]) + ntnb[...]
        sv = jax.nn.relu(t1 + t2)
        h = jax.nn.relu(_dot(sv, rw1[...]) + rb1[...])
        h = jax.nn.relu(_dot(h, rw2[...]) + rb2[...])
        bias = _dot(h, rw3[...]) + rb3[...]
        score = jax.nn.sigmoid(ssumr[...] + bias)
        score_r[...] = score
        pre_r[...] = -jnp.log(score) * avgr[...]

    out = (jax.ShapeDtypeStruct((1, 1), F32),
           jax.ShapeDtypeStruct((1, 1), F32))
    return pl.pallas_call(body, out_shape=out)


def kernel(emb1, edge_index1, edge_attr1, emb2, edge_index2, edge_attr2,
           avg_v, g1_Wself, g1_Wmsg, g1_Wedge, g1_b, g1_Wcross, g2_Wself,
           g2_Wmsg, g2_Wedge, g2_b, g2_Wcross, map_M1, map_M2, cost_M1,
           cost_M2, att_W, ntn_W, ntn_V, ntn_b, reg_W1, reg_b1, reg_W2,
           reg_b2, reg_W3, reg_b3):
    N, D = emb1.shape
    E = edge_index1.shape[1]
    ED = edge_attr1.shape[1]
    F1 = g1_Wself.shape[1]
    F2 = g2_Wself.shape[1]
    T = ntn_b.shape[0]

    src1 = edge_index1[0].astype(jnp.int32)
    dst1 = edge_index1[1].astype(jnp.int32)
    src2 = edge_index2[0].astype(jnp.int32)
    dst2 = edge_index2[1].astype(jnp.int32)
    ea1 = edge_attr1.astype(F32)
    ea2 = edge_attr2.astype(F32)

    zx1 = jnp.zeros((N, F1), F32)
    zx2 = jnp.zeros((N, F2), F32)

    RB = 1000 if N % 1000 == 0 else N
    RE = 8000 if E % 8000 == 0 else E

    xw1a, xw1b, sb1a, sb1b = _tc_pre(N, D, F1, RB)(
        emb1, emb2, g1_Wmsg, g1_Wself)
    ew1a, ew1b = _tc_edge(E, ED, F1, RE)(ea1, ea2, g1_Wedge)
    ax1a, ax1b = _sc_conv(N, E, F1)(
        xw1a, xw1b, src1, dst1, ew1a, src2, dst2, ew1b, zx1)
    h1a, h1b, cs1a, cs1b = _tc_relu_agg(N, F1, RB)(
        ax1a, ax1b, sb1a, sb1b, g1_b.reshape(1, F1))
    e1a, e2a, xw2a, xw2b, sb2a, sb2b = _tc_cross(N, F1, F2, RB, True)(
        h1a, h1b, cs1a, cs1b, g1_Wcross, g2_Wmsg, g2_Wself)

    ew2a, ew2b = _tc_edge(E, ED, F2, RE)(ea1, ea2, g2_Wedge)
    ax2a, ax2b = _sc_conv(N, E, F2)(
        xw2a, xw2b, src1, dst1, ew2a, src2, dst2, ew2b, zx2)
    h2a, h2b, cs2a, cs2b = _tc_relu_agg(N, F2, RB)(
        ax2a, ax2b, sb2a, sb2b, g2_b.reshape(1, F2))
    e1b, e2b = _tc_cross(N, F2, 0, RB, False)(
        h2a, h2b, cs2a, cs2b, g2_Wcross)

    mapping, ssum = _tc_matching(N, F1, F2, 200)(
        e1a, e1b, e2a, e2b, map_M1, map_M2, cost_M1, cost_M2)

    score, pre = _tc_head(N, F2, T)(
        e1b, e2b, att_W, jnp.transpose(ntn_W, (2, 0, 1)),
        jnp.transpose(ntn_V), ntn_b.reshape(1, T), reg_W1,
        reg_b1.reshape(1, -1), reg_W2, reg_b2.reshape(1, -1), reg_W3,
        reg_b3.reshape(1, -1), avg_v.reshape(1, 1).astype(F32), ssum)

    return score.reshape(()), pre.reshape((1,)), mapping


# speedup vs baseline: 4.4585x; 1.0036x over previous
import functools

import jax
import jax.numpy as jnp
from jax import lax
from jax.experimental import pallas as pl
from jax.experimental.pallas import tpu as pltpu
from jax.experimental.pallas import tpu_sc as plsc

F32 = jnp.float32


def _dot(a, b):
    return jnp.dot(a, b, preferred_element_type=F32)


def _dot_t(a, b):
    return lax.dot_general(a, b, (((1,), (1,)), ((), ())),
                           preferred_element_type=F32)


@functools.lru_cache(maxsize=None)
def _tc_pre(N, D, F, R):
    def body(e1, e2, wm, ws, xw1, xw2, sb1, sb2):
        xw1[...] = _dot(e1[...], wm[...])
        xw2[...] = _dot(e2[...], wm[...])
        sb1[...] = _dot(e1[...], ws[...])
        sb2[...] = _dot(e2[...], ws[...])

    out = tuple(jax.ShapeDtypeStruct((N, F), F32) for _ in range(4))
    row = lambda s: pl.BlockSpec((R, s), lambda i: (i, 0))
    full = lambda a, b: pl.BlockSpec((a, b), lambda i: (0, 0))
    return pl.pallas_call(
        body, grid=(N // R,),
        in_specs=[row(D), row(D), full(D, F), full(D, F)],
        out_specs=[row(F)] * 4, out_shape=out)


@functools.lru_cache(maxsize=None)
def _tc_edge(E, ED, F, R):
    def body(a1, a2, we, o1, o2):
        o1[...] = _dot(a1[...], we[...])
        o2[...] = _dot(a2[...], we[...])

    row = lambda s: pl.BlockSpec((R, s), lambda i: (i, 0))
    full = lambda a, b: pl.BlockSpec((a, b), lambda i: (0, 0))
    out = tuple(jax.ShapeDtypeStruct((E, F), F32) for _ in range(2))
    return pl.pallas_call(
        body, grid=(E // R,),
        in_specs=[row(ED), row(ED), full(ED, F)],
        out_specs=[row(F)] * 2, out_shape=out)


@functools.lru_cache(maxsize=None)
def _tc_relu_agg(N, F, R):
    def body(ax1, ax2, sb1, sb2, bb, h1r, h2r, cs1r, cs2r):
        h1 = jax.nn.relu((sb1[...] + ax1[...]) + bb[...])
        h2 = jax.nn.relu((sb2[...] + ax2[...]) + bb[...])
        h1r[...] = h1
        h2r[...] = h2

        @pl.when(pl.program_id(0) == 0)
        def _():
            cs1r[...] = jnp.zeros((1, F), F32)
            cs2r[...] = jnp.zeros((1, F), F32)

        cs1r[...] += jnp.sum(h1, axis=0, keepdims=True)
        cs2r[...] += jnp.sum(h2, axis=0, keepdims=True)

    row = lambda s: pl.BlockSpec((R, s), lambda i: (i, 0))
    full = lambda a, b: pl.BlockSpec((a, b), lambda i: (0, 0))
    out = (jax.ShapeDtypeStruct((N, F), F32),
           jax.ShapeDtypeStruct((N, F), F32),
           jax.ShapeDtypeStruct((1, F), F32),
           jax.ShapeDtypeStruct((1, F), F32))
    return pl.pallas_call(
        body, grid=(N // R,),
        in_specs=[row(F), row(F), row(F), row(F), full(1, F)],
        out_specs=[row(F), row(F), full(1, F), full(1, F)],
        out_shape=out)


@functools.lru_cache(maxsize=None)
def _tc_cross(N, F, FN, R, project):
    def body(*refs):
        if project:
            (h1, h2, cs1, cs2, wc, wm2, ws2,
             o1r, o2r, xw1r, xw2r, nb1r, nb2r) = refs
        else:
            (h1, h2, cs1, cs2, wc, o1r, o2r) = refs
        inv_n = F32(1.0 / N)
        t1 = jnp.tanh(_dot(cs2[...] * inv_n, wc[...]))
        t2 = jnp.tanh(_dot(cs1[...] * inv_n, wc[...]))
        o1 = h1[...] + t1
        o2 = h2[...] + t2
        o1r[...] = o1
        o2r[...] = o2
        if project:
            xw1r[...] = _dot(o1, wm2[...])
            xw2r[...] = _dot(o2, wm2[...])
            nb1r[...] = _dot(o1, ws2[...])
            nb2r[...] = _dot(o2, ws2[...])

    row = lambda s: pl.BlockSpec((R, s), lambda i: (i, 0))
    full = lambda a, b: pl.BlockSpec((a, b), lambda i: (0, 0))
    out = [jax.ShapeDtypeStruct((N, F), F32)] * 2
    in_specs = [row(F), row(F), full(1, F), full(1, F), full(F, F)]
    out_specs = [row(F), row(F)]
    if project:
        out += [jax.ShapeDtypeStruct((N, FN), F32)] * 4
        in_specs += [full(F, FN), full(F, FN)]
        out_specs += [row(FN)] * 4
    return pl.pallas_call(body, grid=(N // R,), in_specs=in_specs,
                          out_specs=out_specs, out_shape=tuple(out))


@functools.lru_cache(maxsize=None)
def _sc_conv(N, E, F):
    NS = 16
    EPT = E // NS
    CH = 128
    n_main = EPT // CH
    TAIL = EPT - n_main * CH
    RPT = (N // NS) // 8 * 8
    RTAIL = N - RPT * NS

    NBUF = 5
    n_grp = n_main // NBUF
    n_rem = n_main - n_grp * NBUF

    mesh = plsc.VectorSubcoreMesh(core_axis_name="c", subcore_axis_name="s",
                                  num_cores=2, num_subcores=16)

    outs = (jax.ShapeDtypeStruct((N, F), F32),
            jax.ShapeDtypeStruct((N, F), F32))

    scratch = [pltpu.VMEM_SHARED((N, F), F32)]
    for _ in range(NBUF):
        scratch += [pltpu.VMEM((CH,), jnp.int32), pltpu.VMEM((CH,), jnp.int32),
                    pltpu.VMEM((CH, F), F32), pltpu.VMEM((CH, F), F32)]
    if TAIL:
        scratch += [pltpu.VMEM((TAIL,), jnp.int32),
                    pltpu.VMEM((TAIL,), jnp.int32),
                    pltpu.VMEM((TAIL, F), F32), pltpu.VMEM((TAIL, F), F32)]
    scratch += [pltpu.SemaphoreType.DMA] * (2 * NBUF)

    def body(*refs):
        it = iter(refs)
        xw1, xw2 = next(it), next(it)
        src1, dst1, ew1 = next(it), next(it), next(it)
        src2, dst2, ew2 = next(it), next(it), next(it)
        zx = next(it)
        aggx1, aggx2 = next(it), next(it)
        aggx_sp = next(it)
        bufs = [(next(it), next(it), next(it), next(it))
                for _ in range(NBUF)]
        if TAIL:
            tailb = (next(it), next(it), next(it), next(it))
        sems = [(next(it), next(it)) for _ in range(NBUF)]

        c = lax.axis_index("c")
        s = lax.axis_index("s")
        r0 = s * RPT

        def sliced_copy(src_ref, dst_ref):
            pltpu.sync_copy(src_ref.at[pl.ds(r0, RPT)],
                            dst_ref.at[pl.ds(r0, RPT)])
            if RTAIL:
                @pl.when(s == 0)
                def _():
                    pltpu.sync_copy(src_ref.at[pl.ds(RPT * NS, RTAIL)],
                                    dst_ref.at[pl.ds(RPT * NS, RTAIL)])

        sliced_copy(zx, aggx_sp)
        plsc.subcore_barrier()

        def run(src, dst, ew, xw):
            base = s * EPT

            def start(off, buf, sem2, ch):
                isb, idb, rb, rb2 = buf
                sg, se = sem2
                pltpu.sync_copy(src.at[pl.ds(off, ch)], isb)
                pltpu.sync_copy(dst.at[pl.ds(off, ch)], idb)
                he = pltpu.async_copy(ew.at[pl.ds(off, ch)], rb, se)
                return he

            def gadd(he, buf, sem2):
                isb, idb, rb, rb2 = buf
                sg, se = sem2
                he.wait()
                return pltpu.async_copy(xw.at[isb], rb, sg, add=True)

            def finish(hg, buf):
                isb, idb, rb, rb2 = buf
                hg.wait()
                pltpu.sync_copy(rb, aggx_sp.at[idb], add=True)

            def bodyf(g, carry):
                goff = base + g * (NBUF * CH)
                hes = [start(goff + b * CH, bufs[b], sems[b], CH)
                       for b in range(NBUF)]
                hgs = [gadd(hes[b], bufs[b], sems[b]) for b in range(NBUF)]
                for b in range(NBUF):
                    finish(hgs[b], bufs[b])
                return carry

            lax.fori_loop(0, n_grp, bodyf, 0)
            for j in range(n_rem):
                off = base + (n_grp * NBUF + j) * CH
                he = start(off, bufs[0], sems[0], CH)
                finish(gadd(he, bufs[0], sems[0]), bufs[0])
            if TAIL:
                he = start(base + n_main * CH, tailb, sems[0], TAIL)
                finish(gadd(he, tailb, sems[0]), tailb)

        @pl.when(c == 0)
        def _():
            run(src1, dst1, ew1, xw1)

        @pl.when(c == 1)
        def _():
            run(src2, dst2, ew2, xw2)

        plsc.subcore_barrier()

        @pl.when(c == 0)
        def _():
            sliced_copy(aggx_sp, aggx1)

        @pl.when(c == 1)
        def _():
            sliced_copy(aggx_sp, aggx2)

    return pl.kernel(body, out_type=outs, mesh=mesh,
                     scratch_types=tuple(scratch),
                     compiler_params=pltpu.CompilerParams(
                         use_tc_tiling_on_sc=False))


@functools.lru_cache(maxsize=None)
def _tc_matching(N, F1, F2, R):
    grid = (N // R,)

    def body(a1r, b1r, q1r, q2r, mm1, mm2, cm1, cm2, map_r, ssum_r):
        a1 = a1r[...]
        b1 = b1r[...]
        q1 = q1r[...]
        q2 = q2r[...]
        lm = _dot_t(_dot(a1, mm1[...]), q1) + _dot_t(_dot(b1, mm2[...]), q2)
        mx = jnp.max(lm, axis=1, keepdims=True)
        ex = jnp.exp(lm - mx)
        ssx = jnp.sum(ex, axis=1, keepdims=True)
        mapping = ex * (1.0 / ssx)
        map_r[...] = mapping
        lc = _dot_t(_dot(a1, cm1[...]), q1) + _dot_t(_dot(b1, cm2[...]), q2)
        partial = jnp.sum(mapping * lc)

        @pl.when(pl.program_id(0) == 0)
        def _():
            ssum_r[...] = jnp.zeros((1, 1), F32)

        ssum_r[...] += jnp.reshape(partial, (1, 1))

    return pl.pallas_call(
        body,
        grid=grid,
        in_specs=[
            pl.BlockSpec((R, F1), lambda i: (i, 0)),
            pl.BlockSpec((R, F2), lambda i: (i, 0)),
            pl.BlockSpec((N, F1), lambda i: (0, 0)),
            pl.BlockSpec((N, F2), lambda i: (0, 0)),
            pl.BlockSpec((F1, F1), lambda i: (0, 0)),
            pl.BlockSpec((F2, F2), lambda i: (0, 0)),
            pl.BlockSpec((F1, F1), lambda i: (0, 0)),
            pl.BlockSpec((F2, F2), lambda i: (0, 0)),
        ],
        out_specs=[
            pl.BlockSpec((R, N), lambda i: (i, 0)),
            pl.BlockSpec((1, 1), lambda i: (0, 0)),
        ],
        out_shape=[
            jax.ShapeDtypeStruct((N, N), F32),
            jax.ShapeDtypeStruct((1, 1), F32),
        ],
    )


@functools.lru_cache(maxsize=None)
def _tc_head(N, F2, T):
    def body(h1r, h2r, attw, ntnw, ntnv, ntnb, rw1, rb1, rw2, rb2, rw3, rb3,
             avgr, ssumr, score_r, pre_r):
        def attpool(h):
            m = jnp.mean(h, axis=0, keepdims=True)
            ctx = jnp.tanh(_dot(m, attw[...]))
            a = jax.nn.sigmoid(jnp.sum(h * ctx, axis=1, keepdims=True))
            return jnp.sum(h * a, axis=0, keepdims=True)

        ge1 = attpool(h1r[...])
        ge2 = attpool(h2r[...])
        kiota = lax.broadcasted_iota(jnp.int32, (1, T), 1)
        t1 = jnp.zeros((1, T), F32)
        for k in range(T):
            u = _dot(ge1, ntnw[k])
            sk = jnp.sum(u * ge2)
            t1 = t1 + jnp.where(kiota == k, sk, 0.0)
        cat = jnp.concatenate([ge1, ge2], axis=1)
        t2 = _dot(cat, ntnv[...]) + ntnb[...]
        sv = jax.nn.relu(t1 + t2)
        h = jax.nn.relu(_dot(sv, rw1[...]) + rb1[...])
        h = jax.nn.relu(_dot(h, rw2[...]) + rb2[...])
        bias = _dot(h, rw3[...]) + rb3[...]
        score = jax.nn.sigmoid(ssumr[...] + bias)
        score_r[...] = score
        pre_r[...] = -jnp.log(score) * avgr[...]

    out = (jax.ShapeDtypeStruct((1, 1), F32),
           jax.ShapeDtypeStruct((1, 1), F32))
    return pl.pallas_call(body, out_shape=out)


def kernel(emb1, edge_index1, edge_attr1, emb2, edge_index2, edge_attr2,
           avg_v, g1_Wself, g1_Wmsg, g1_Wedge, g1_b, g1_Wcross, g2_Wself,
           g2_Wmsg, g2_Wedge, g2_b, g2_Wcross, map_M1, map_M2, cost_M1,
           cost_M2, att_W, ntn_W, ntn_V, ntn_b, reg_W1, reg_b1, reg_W2,
           reg_b2, reg_W3, reg_b3):
    N, D = emb1.shape
    E = edge_index1.shape[1]
    ED = edge_attr1.shape[1]
    F1 = g1_Wself.shape[1]
    F2 = g2_Wself.shape[1]
    T = ntn_b.shape[0]

    src1 = edge_index1[0].astype(jnp.int32)
    dst1 = edge_index1[1].astype(jnp.int32)
    src2 = edge_index2[0].astype(jnp.int32)
    dst2 = edge_index2[1].astype(jnp.int32)
    ea1 = edge_attr1.astype(F32)
    ea2 = edge_attr2.astype(F32)

    zx1 = jnp.zeros((N, F1), F32)
    zx2 = jnp.zeros((N, F2), F32)

    RB = 1000 if N % 1000 == 0 else N
    RE = 8000 if E % 8000 == 0 else E

    xw1a, xw1b, sb1a, sb1b = _tc_pre(N, D, F1, RB)(
        emb1, emb2, g1_Wmsg, g1_Wself)
    ew1a, ew1b = _tc_edge(E, ED, F1, RE)(ea1, ea2, g1_Wedge)
    ax1a, ax1b = _sc_conv(N, E, F1)(
        xw1a, xw1b, src1, dst1, ew1a, src2, dst2, ew1b, zx1)
    h1a, h1b, cs1a, cs1b = _tc_relu_agg(N, F1, RB)(
        ax1a, ax1b, sb1a, sb1b, g1_b.reshape(1, F1))
    e1a, e2a, xw2a, xw2b, sb2a, sb2b = _tc_cross(N, F1, F2, RB, True)(
        h1a, h1b, cs1a, cs1b, g1_Wcross, g2_Wmsg, g2_Wself)

    ew2a, ew2b = _tc_edge(E, ED, F2, RE)(ea1, ea2, g2_Wedge)
    ax2a, ax2b = _sc_conv(N, E, F2)(
        xw2a, xw2b, src1, dst1, ew2a, src2, dst2, ew2b, zx2)
    h2a, h2b, cs2a, cs2b = _tc_relu_agg(N, F2, RB)(
        ax2a, ax2b, sb2a, sb2b, g2_b.reshape(1, F2))
    e1b, e2b = _tc_cross(N, F2, 0, RB, False)(
        h2a, h2b, cs2a, cs2b, g2_Wcross)

    mapping, ssum = _tc_matching(N, F1, F2, 200)(
        e1a, e1b, e2a, e2b, map_M1, map_M2, cost_M1, cost_M2)

    score, pre = _tc_head(N, F2, T)(
        e1b, e2b, att_W, jnp.transpose(ntn_W, (2, 0, 1)),
        jnp.transpose(ntn_V), ntn_b.reshape(1, T), reg_W1,
        reg_b1.reshape(1, -1), reg_W2, reg_b2.reshape(1, -1), reg_W3,
        reg_b3.reshape(1, -1), avg_v.reshape(1, 1).astype(F32), ssum)

    return score.reshape(()), pre.reshape((1,)), mapping
